# Initial kernel scaffold; baseline (speedup 1.0000x reference)
import jax
import jax.numpy as jnp
from jax.experimental import pallas as pl


def kernel(query_features, query_stock_ids, batch_stock_ids, batch_stock_features, stock_table, pw1, pb1, pw2, pb2, qw, qb, kw, kb, vw, vb, ow, ob, gw1, gb1, gw2, gb2, ln_g, ln_b):
    raise NotImplementedError("write your pallas kernel here")


# speedup vs baseline: 16.7438x; 16.7438x over previous
import functools
import math

import jax
import jax.numpy as jnp
from jax import lax
from jax.experimental import pallas as pl
from jax.experimental.pallas import tpu as pltpu
from jax.experimental.pallas import tpu_sc as plsc

B, S, H = 8, 32, 1024
HEADS, DH = 16, 64
NSTOCK, SED = 6000, 256
NKV, TOPK = 512, 128

NIDS = B + B * NKV
NIDS_PAD = 4352
ROWS = HEADS * S


@functools.lru_cache(maxsize=1)
def _make_sc_gather():
    info = plsc.get_sparse_core_info()
    nw = info.num_cores * info.num_subcores
    b_per_w = NIDS_PAD // nw
    mesh = plsc.VectorSubcoreMesh(core_axis_name="c", subcore_axis_name="s")

    @functools.partial(
        pl.kernel,
        mesh=mesh,
        out_type=jax.ShapeDtypeStruct((NIDS_PAD, SED), jnp.float32),
        scratch_types=[
            pltpu.VMEM((b_per_w,), jnp.int32),
            pltpu.VMEM((b_per_w, SED), jnp.float32),
            pltpu.SemaphoreType.DMA,
        ],
    )
    def gather_kernel(table_hbm, idx_hbm, out_hbm, idx_v, rows_v, sem):
        wid = lax.axis_index("s") * info.num_cores + lax.axis_index("c")
        base = wid * b_per_w
        pltpu.sync_copy(idx_hbm.at[pl.ds(base, b_per_w)], idx_v)
        pltpu.async_copy(table_hbm.at[idx_v], rows_v, sem).wait()
        pltpu.sync_copy(rows_v, out_hbm.at[pl.ds(base, b_per_w)])

    return gather_kernel


def _gather_rows(table, ids):
    return _make_sc_gather()(table, ids)


def _embed_mlp_body(e_ref, pw1_ref, pb1_ref, pw2_ref, pb2_ref, out_ref):
    h1 = jax.nn.gelu(jnp.dot(e_ref[...], pw1_ref[...]) + pb1_ref[...])
    out_ref[...] = jnp.dot(h1, pw2_ref[...]) + pb2_ref[...]


def _embed_mlp(e_all, pw1, pb1r, pw2, pb2r):
    nblk = NIDS_PAD // 256
    return pl.pallas_call(
        _embed_mlp_body,
        grid=(nblk,),
        in_specs=[
            pl.BlockSpec((256, SED), lambda i: (i, 0)),
            pl.BlockSpec((SED, H), lambda i: (0, 0)),
            pl.BlockSpec((1, H), lambda i: (0, 0)),
            pl.BlockSpec((H, H), lambda i: (0, 0)),
            pl.BlockSpec((1, H), lambda i: (0, 0)),
        ],
        out_specs=pl.BlockSpec((256, H), lambda i: (i, 0)),
        out_shape=jax.ShapeDtypeStruct((NIDS_PAD, H), jnp.float32),
    )(e_all, pw1, pb1r, pw2, pb2r)


def _proj_body(qf_ref, bsf_ref, bse_ref, qse_ref, qw_ref, qb_ref, kw_ref,
               kb_ref, vw_ref, vb_ref, gw1a_ref, gw1b_ref, gb1_ref, gw2_ref,
               gb2_ref, v_out, gs_out):
    kv = bsf_ref[0] + bse_ref[0]
    k = jnp.dot(kv, kw_ref[...]) + kb_ref[...]
    v_out[0] = jnp.dot(kv, vw_ref[...]) + vb_ref[...]
    qf = qf_ref[0]
    qc = jnp.mean(qf, axis=0, keepdims=True) + qse_ref[0]
    q = jnp.dot(qf, qw_ref[...]) + qb_ref[...]
    qt = q.T
    g1 = jnp.dot(qc, gw1a_ref[...]) + jnp.dot(kv, gw1b_ref[...]) + gb1_ref[...]
    g = jax.nn.sigmoid(jnp.dot(jax.nn.gelu(g1), gw2_ref[...]) + gb2_ref[...])
    scale = 1.0 / math.sqrt(DH)
    for h in range(HEADS):
        kh = k[:, h * DH:(h + 1) * DH]
        qh_t = qt[h * DH:(h + 1) * DH, :]
        sh_t = jnp.dot(kh, qh_t) * scale * g[:, h:h + 1]
        gs_out[0, :, h * S:(h + 1) * S] = sh_t


def _proj(qf, bsf, bse, qse, qw, qbr, kw, kbr, vw, vbr, gw1a, gw1b, gb1r,
          gw2, gb2r):
    return pl.pallas_call(
        _proj_body,
        grid=(B,),
        in_specs=[
            pl.BlockSpec((1, S, H), lambda b: (b, 0, 0)),
            pl.BlockSpec((1, NKV, H), lambda b: (b, 0, 0)),
            pl.BlockSpec((1, NKV, H), lambda b: (b, 0, 0)),
            pl.BlockSpec((1, 1, H), lambda b: (b, 0, 0)),
            pl.BlockSpec((H, H), lambda b: (0, 0)),
            pl.BlockSpec((1, H), lambda b: (0, 0)),
            pl.BlockSpec((H, H), lambda b: (0, 0)),
            pl.BlockSpec((1, H), lambda b: (0, 0)),
            pl.BlockSpec((H, H), lambda b: (0, 0)),
            pl.BlockSpec((1, H), lambda b: (0, 0)),
            pl.BlockSpec((H, H), lambda b: (0, 0)),
            pl.BlockSpec((H, H), lambda b: (0, 0)),
            pl.BlockSpec((1, H), lambda b: (0, 0)),
            pl.BlockSpec((H, HEADS), lambda b: (0, 0)),
            pl.BlockSpec((1, HEADS), lambda b: (0, 0)),
        ],
        out_specs=[
            pl.BlockSpec((1, NKV, H), lambda b: (b, 0, 0)),
            pl.BlockSpec((1, NKV, ROWS), lambda b: (b, 0, 0)),
        ],
        out_shape=[
            jax.ShapeDtypeStruct((B, NKV, H), jnp.float32),
            jax.ShapeDtypeStruct((B, NKV, ROWS), jnp.float32),
        ],
    )(qf, bsf, bse, qse, qw, qbr, kw, kbr, vw, vbr, gw1a, gw1b, gb1r, gw2,
      gb2r)


def _attend_body(gs_ref, v_ref, qf_ref, ow_ref, ob_ref, lng_ref, lnb_ref,
                 y_out, attn_out, idx_out, sorted_ref, idxs_ref, ctx_ref):
    sc = gs_ref[0]
    sub_iota = lax.broadcasted_iota(jnp.int32, (NKV, ROWS), 0)

    def step(j, carry):
        pv, pi = carry
        rem = (sc < pv) | ((sc == pv) & (sub_iota > pi))
        cur = jnp.max(jnp.where(rem, sc, -jnp.inf), axis=0, keepdims=True)
        hit = rem & (sc == cur)
        idxv = jnp.min(jnp.where(hit, sub_iota, jnp.int32(1 << 30)), axis=0,
                       keepdims=True)
        sorted_ref[pl.ds(j, 1), :] = cur
        idxs_ref[pl.ds(j, 1), :] = idxv
        return cur, idxv

    lax.fori_loop(0, TOPK, step,
                  (jnp.full((1, ROWS), jnp.inf, jnp.float32),
                   jnp.full((1, ROWS), -1, jnp.int32)))

    st = sorted_ref[...]
    v0 = st[0:1, :]
    e = jnp.exp(st - v0)
    denom = jnp.sum(e, axis=0, keepdims=True)
    attn_out[0] = (e / denom).T
    idx_out[0] = idxs_ref[...].T

    th = st[TOPK - 1:TOPK, :]
    th_i = idxs_ref[TOPK - 1:TOPK, :]
    sel = (sc > th) | ((sc == th) & (sub_iota <= th_i))
    p_t = jnp.where(sel, jnp.exp(sc - v0), 0.0) / denom
    p = p_t.T
    vv = v_ref[0]
    for h in range(HEADS):
        ph = p[h * S:(h + 1) * S, :]
        vh = vv[:, h * DH:(h + 1) * DH]
        ctx_ref[:, h * DH:(h + 1) * DH] = jnp.dot(ph, vh)

    out = jnp.dot(ctx_ref[...], ow_ref[...]) + ob_ref[...]
    x = qf_ref[0] + out
    mu = jnp.mean(x, axis=1, keepdims=True)
    var = jnp.mean((x - mu) ** 2, axis=1, keepdims=True)
    y_out[0] = (x - mu) * lax.rsqrt(var + 1e-5) * lng_ref[...] + lnb_ref[...]


def _attend(gs, v, qf, ow, obr, lngr, lnbr):
    return pl.pallas_call(
        _attend_body,
        grid=(B,),
        in_specs=[
            pl.BlockSpec((1, NKV, ROWS), lambda b: (b, 0, 0)),
            pl.BlockSpec((1, NKV, H), lambda b: (b, 0, 0)),
            pl.BlockSpec((1, S, H), lambda b: (b, 0, 0)),
            pl.BlockSpec((H, H), lambda b: (0, 0)),
            pl.BlockSpec((1, H), lambda b: (0, 0)),
            pl.BlockSpec((1, H), lambda b: (0, 0)),
            pl.BlockSpec((1, H), lambda b: (0, 0)),
        ],
        out_specs=[
            pl.BlockSpec((1, S, H), lambda b: (b, 0, 0)),
            pl.BlockSpec((1, ROWS, TOPK), lambda b: (b, 0, 0)),
            pl.BlockSpec((1, ROWS, TOPK), lambda b: (b, 0, 0)),
        ],
        out_shape=[
            jax.ShapeDtypeStruct((B, S, H), jnp.float32),
            jax.ShapeDtypeStruct((B, ROWS, TOPK), jnp.float32),
            jax.ShapeDtypeStruct((B, ROWS, TOPK), jnp.int32),
        ],
        scratch_shapes=[
            pltpu.VMEM((TOPK, ROWS), jnp.float32),
            pltpu.VMEM((TOPK, ROWS), jnp.int32),
            pltpu.VMEM((S, H), jnp.float32),
        ],
    )(gs, v, qf, ow, obr, lngr, lnbr)


def kernel(query_features, query_stock_ids, batch_stock_ids,
           batch_stock_features, stock_table, pw1, pb1, pw2, pb2, qw, qb, kw,
           kb, vw, vb, ow, ob, gw1, gb1, gw2, gb2, ln_g, ln_b):
    ids_all = jnp.concatenate([
        query_stock_ids.astype(jnp.int32),
        batch_stock_ids.reshape(-1).astype(jnp.int32),
        jnp.zeros((NIDS_PAD - NIDS,), jnp.int32),
    ])
    e_all = _gather_rows(stock_table, ids_all)
    emb = _embed_mlp(e_all, pw1, pb1.reshape(1, H), pw2, pb2.reshape(1, H))
    qse = emb[:B].reshape(B, 1, H)
    bse = emb[B:B + B * NKV].reshape(B, NKV, H)

    v_all, gs = _proj(
        query_features, batch_stock_features, bse, qse,
        qw, qb.reshape(1, H), kw, kb.reshape(1, H), vw, vb.reshape(1, H),
        gw1[:H], gw1[H:], gb1.reshape(1, H), gw2, gb2.reshape(1, HEADS))

    y, attn, idx = _attend(
        gs, v_all, query_features, ow, ob.reshape(1, H),
        ln_g.reshape(1, H), ln_b.reshape(1, H))

    attn = attn.reshape(B, HEADS, S, TOPK)
    attended = idx.reshape(B, HEADS, S, TOPK)[:, :, S - 1, :]
    return y, attn, attended


# speedup vs baseline: 19.2835x; 1.1517x over previous
import functools
import math

import jax
import jax.numpy as jnp
from jax import lax
from jax.experimental import pallas as pl
from jax.experimental.pallas import tpu as pltpu
from jax.experimental.pallas import tpu_sc as plsc

B, S, H = 8, 32, 1024
HEADS, DH = 16, 64
NSTOCK, SED = 6000, 256
NKV, TOPK = 512, 128

NIDS = B + B * NKV
NIDS_PAD = 4352
ROWS = HEADS * S


@functools.lru_cache(maxsize=1)
def _make_sc_gather():
    info = plsc.get_sparse_core_info()
    nw = info.num_cores * info.num_subcores
    b_per_w = NIDS_PAD // nw
    mesh = plsc.VectorSubcoreMesh(core_axis_name="c", subcore_axis_name="s")

    @functools.partial(
        pl.kernel,
        mesh=mesh,
        out_type=jax.ShapeDtypeStruct((NIDS_PAD, SED), jnp.float32),
        scratch_types=[
            pltpu.VMEM((b_per_w,), jnp.int32),
            pltpu.VMEM((b_per_w, SED), jnp.float32),
            pltpu.SemaphoreType.DMA,
        ],
    )
    def gather_kernel(table_hbm, idx_hbm, out_hbm, idx_v, rows_v, sem):
        wid = lax.axis_index("s") * info.num_cores + lax.axis_index("c")
        base = wid * b_per_w
        pltpu.sync_copy(idx_hbm.at[pl.ds(base, b_per_w)], idx_v)
        pltpu.async_copy(table_hbm.at[idx_v], rows_v, sem).wait()
        pltpu.sync_copy(rows_v, out_hbm.at[pl.ds(base, b_per_w)])

    return gather_kernel


def _gather_rows(table, ids):
    return _make_sc_gather()(table, ids)


def _embed_mlp_body(e_ref, pw1_ref, pb1_ref, pw2_ref, pb2_ref, out_ref):
    h1 = jax.nn.gelu(jnp.dot(e_ref[...], pw1_ref[...]) + pb1_ref[...])
    out_ref[...] = jnp.dot(h1, pw2_ref[...]) + pb2_ref[...]


def _embed_mlp(e_all, pw1, pb1r, pw2, pb2r):
    nblk = NIDS_PAD // 256
    return pl.pallas_call(
        _embed_mlp_body,
        grid=(nblk,),
        in_specs=[
            pl.BlockSpec((256, SED), lambda i: (i, 0)),
            pl.BlockSpec((SED, H), lambda i: (0, 0)),
            pl.BlockSpec((1, H), lambda i: (0, 0)),
            pl.BlockSpec((H, H), lambda i: (0, 0)),
            pl.BlockSpec((1, H), lambda i: (0, 0)),
        ],
        out_specs=pl.BlockSpec((256, H), lambda i: (i, 0)),
        out_shape=jax.ShapeDtypeStruct((NIDS_PAD, H), jnp.float32),
    )(e_all, pw1, pb1r, pw2, pb2r)


def _proj_body(qf_ref, bsf_ref, bse_ref, qse_ref, qw_ref, qb_ref, kw_ref,
               kb_ref, vw_ref, vb_ref, gw1a_ref, gw1b_ref, gb1_ref, gw2_ref,
               gb2_ref, v_out, gs_out, gs31_out):
    kv = bsf_ref[0] + bse_ref[0]
    k = jnp.dot(kv, kw_ref[...]) + kb_ref[...]
    v_out[0] = jnp.dot(kv, vw_ref[...]) + vb_ref[...]
    qf = qf_ref[0]
    qc = jnp.mean(qf, axis=0, keepdims=True) + qse_ref[0]
    q = jnp.dot(qf, qw_ref[...]) + qb_ref[...]
    qt = q.T
    g1 = jnp.dot(qc, gw1a_ref[...]) + jnp.dot(kv, gw1b_ref[...]) + gb1_ref[...]
    g = jax.nn.sigmoid(jnp.dot(jax.nn.gelu(g1), gw2_ref[...]) + gb2_ref[...])
    scale = 1.0 / math.sqrt(DH)
    for h in range(HEADS):
        kh = k[:, h * DH:(h + 1) * DH]
        qh_t = qt[h * DH:(h + 1) * DH, :]
        sh_t = jnp.dot(kh, qh_t) * scale * g[:, h:h + 1]
        gs_out[0, :, h * S:(h + 1) * S] = sh_t
        gs31_out[0, :, h:h + 1] = sh_t[:, S - 1:S]


def _proj(qf, bsf, bse, qse, qw, qbr, kw, kbr, vw, vbr, gw1a, gw1b, gb1r,
          gw2, gb2r):
    return pl.pallas_call(
        _proj_body,
        grid=(B,),
        in_specs=[
            pl.BlockSpec((1, S, H), lambda b: (b, 0, 0)),
            pl.BlockSpec((1, NKV, H), lambda b: (b, 0, 0)),
            pl.BlockSpec((1, NKV, H), lambda b: (b, 0, 0)),
            pl.BlockSpec((1, 1, H), lambda b: (b, 0, 0)),
            pl.BlockSpec((H, H), lambda b: (0, 0)),
            pl.BlockSpec((1, H), lambda b: (0, 0)),
            pl.BlockSpec((H, H), lambda b: (0, 0)),
            pl.BlockSpec((1, H), lambda b: (0, 0)),
            pl.BlockSpec((H, H), lambda b: (0, 0)),
            pl.BlockSpec((1, H), lambda b: (0, 0)),
            pl.BlockSpec((H, H), lambda b: (0, 0)),
            pl.BlockSpec((H, H), lambda b: (0, 0)),
            pl.BlockSpec((1, H), lambda b: (0, 0)),
            pl.BlockSpec((H, HEADS), lambda b: (0, 0)),
            pl.BlockSpec((1, HEADS), lambda b: (0, 0)),
        ],
        out_specs=[
            pl.BlockSpec((1, NKV, H), lambda b: (b, 0, 0)),
            pl.BlockSpec((1, NKV, ROWS), lambda b: (b, 0, 0)),
            pl.BlockSpec((1, NKV, HEADS), lambda b: (b, 0, 0)),
        ],
        out_shape=[
            jax.ShapeDtypeStruct((B, NKV, H), jnp.float32),
            jax.ShapeDtypeStruct((B, NKV, ROWS), jnp.float32),
            jax.ShapeDtypeStruct((B, NKV, HEADS), jnp.float32),
        ],
    )(qf, bsf, bse, qse, qw, qbr, kw, kbr, vw, vbr, gw1a, gw1b, gb1r, gw2,
      gb2r)


def _attend_body(gs_ref, gs31_ref, v_ref, qf_ref, ow_ref, ob_ref, lng_ref,
                 lnb_ref, y_out, attn_out, idx31_out, sorted_ref, idx31_ref,
                 ctx_ref):
    sc = gs_ref[0]
    sc31 = gs31_ref[0]
    iota31 = lax.broadcasted_iota(jnp.int32, (NKV, HEADS), 0)

    def step(j, carry):
        pv, c, pv31, pi31 = carry
        lt = sc < pv
        eqc = jnp.sum((sc == pv).astype(jnp.float32), axis=0, keepdims=True)
        nxt = jnp.max(jnp.where(lt, sc, -jnp.inf), axis=0, keepdims=True)
        more = c < eqc
        newv = jnp.where(more, pv, nxt)
        newc = jnp.where(more, c + 1.0, 1.0)
        sorted_ref[pl.ds(j, 1), :] = newv

        rem = (sc31 < pv31) | ((sc31 == pv31) & (iota31 > pi31))
        cur = jnp.max(jnp.where(rem, sc31, -jnp.inf), axis=0, keepdims=True)
        hit = rem & (sc31 == cur)
        idxv = jnp.min(jnp.where(hit, iota31, jnp.int32(1 << 30)), axis=0,
                       keepdims=True)
        idx31_ref[pl.ds(j, 1), :] = idxv
        return newv, newc, cur, idxv

    lax.fori_loop(0, TOPK, step,
                  (jnp.full((1, ROWS), jnp.inf, jnp.float32),
                   jnp.ones((1, ROWS), jnp.float32),
                   jnp.full((1, HEADS), jnp.inf, jnp.float32),
                   jnp.full((1, HEADS), -1, jnp.int32)))

    st = sorted_ref[...]
    v0 = st[0:1, :]
    e = jnp.exp(st - v0)
    denom = jnp.sum(e, axis=0, keepdims=True)
    attn_out[0] = (e / denom).T
    idx31_out[0] = idx31_ref[...].T

    th = st[TOPK - 1:TOPK, :]
    gt = sc > th
    eqth = (sc == th).astype(jnp.float32)
    k_need = TOPK - jnp.sum(gt.astype(jnp.float32), axis=0, keepdims=True)
    rank = eqth
    for shift in (1, 2, 4, 8, 16, 32, 64, 128, 256):
        shifted = jnp.concatenate(
            [jnp.zeros((shift, ROWS), jnp.float32), rank[:NKV - shift]],
            axis=0)
        rank = rank + shifted
    sel = gt | ((eqth > 0.0) & (rank <= k_need))
    p_t = jnp.where(sel, jnp.exp(sc - v0), 0.0) / denom
    p = p_t.T
    vv = v_ref[0]
    for h in range(HEADS):
        ph = p[h * S:(h + 1) * S, :]
        vh = vv[:, h * DH:(h + 1) * DH]
        ctx_ref[:, h * DH:(h + 1) * DH] = jnp.dot(ph, vh)

    out = jnp.dot(ctx_ref[...], ow_ref[...]) + ob_ref[...]
    x = qf_ref[0] + out
    mu = jnp.mean(x, axis=1, keepdims=True)
    var = jnp.mean((x - mu) ** 2, axis=1, keepdims=True)
    y_out[0] = (x - mu) * lax.rsqrt(var + 1e-5) * lng_ref[...] + lnb_ref[...]


def _attend(gs, gs31, v, qf, ow, obr, lngr, lnbr):
    return pl.pallas_call(
        _attend_body,
        grid=(B,),
        in_specs=[
            pl.BlockSpec((1, NKV, ROWS), lambda b: (b, 0, 0)),
            pl.BlockSpec((1, NKV, HEADS), lambda b: (b, 0, 0)),
            pl.BlockSpec((1, NKV, H), lambda b: (b, 0, 0)),
            pl.BlockSpec((1, S, H), lambda b: (b, 0, 0)),
            pl.BlockSpec((H, H), lambda b: (0, 0)),
            pl.BlockSpec((1, H), lambda b: (0, 0)),
            pl.BlockSpec((1, H), lambda b: (0, 0)),
            pl.BlockSpec((1, H), lambda b: (0, 0)),
        ],
        out_specs=[
            pl.BlockSpec((1, S, H), lambda b: (b, 0, 0)),
            pl.BlockSpec((1, ROWS, TOPK), lambda b: (b, 0, 0)),
            pl.BlockSpec((1, HEADS, TOPK), lambda b: (b, 0, 0)),
        ],
        out_shape=[
            jax.ShapeDtypeStruct((B, S, H), jnp.float32),
            jax.ShapeDtypeStruct((B, ROWS, TOPK), jnp.float32),
            jax.ShapeDtypeStruct((B, HEADS, TOPK), jnp.int32),
        ],
        scratch_shapes=[
            pltpu.VMEM((TOPK, ROWS), jnp.float32),
            pltpu.VMEM((TOPK, HEADS), jnp.int32),
            pltpu.VMEM((S, H), jnp.float32),
        ],
    )(gs, gs31, v, qf, ow, obr, lngr, lnbr)


def kernel(query_features, query_stock_ids, batch_stock_ids,
           batch_stock_features, stock_table, pw1, pb1, pw2, pb2, qw, qb, kw,
           kb, vw, vb, ow, ob, gw1, gb1, gw2, gb2, ln_g, ln_b):
    ids_all = jnp.concatenate([
        query_stock_ids.astype(jnp.int32),
        batch_stock_ids.reshape(-1).astype(jnp.int32),
        jnp.zeros((NIDS_PAD - NIDS,), jnp.int32),
    ])
    e_all = _gather_rows(stock_table, ids_all)
    emb = _embed_mlp(e_all, pw1, pb1.reshape(1, H), pw2, pb2.reshape(1, H))
    qse = emb[:B].reshape(B, 1, H)
    bse = emb[B:B + B * NKV].reshape(B, NKV, H)

    v_all, gs, gs31 = _proj(
        query_features, batch_stock_features, bse, qse,
        qw, qb.reshape(1, H), kw, kb.reshape(1, H), vw, vb.reshape(1, H),
        gw1[:H], gw1[H:], gb1.reshape(1, H), gw2, gb2.reshape(1, HEADS))

    y, attn, attended = _attend(
        gs, gs31, v_all, query_features, ow, ob.reshape(1, H),
        ln_g.reshape(1, H), ln_b.reshape(1, H))

    attn = attn.reshape(B, HEADS, S, TOPK)
    return y, attn, attended


# speedup vs baseline: 22.4854x; 1.1660x over previous
import functools
import math

import jax
import jax.numpy as jnp
from jax import lax
from jax.experimental import pallas as pl
from jax.experimental.pallas import tpu as pltpu
from jax.experimental.pallas import tpu_sc as plsc

B, S, H = 8, 32, 1024
HEADS, DH = 16, 64
NSTOCK, SED = 6000, 256
NKV, TOPK = 512, 128

NIDS = B + B * NKV
NIDS_PAD = 4352
ROWS = HEADS * S


@functools.lru_cache(maxsize=1)
def _make_sc_gather():
    info = plsc.get_sparse_core_info()
    nw = info.num_cores * info.num_subcores
    b_per_w = NIDS_PAD // nw
    mesh = plsc.VectorSubcoreMesh(core_axis_name="c", subcore_axis_name="s")

    @functools.partial(
        pl.kernel,
        mesh=mesh,
        out_type=jax.ShapeDtypeStruct((NIDS_PAD, SED), jnp.float32),
        scratch_types=[
            pltpu.VMEM((b_per_w,), jnp.int32),
            pltpu.VMEM((b_per_w, SED), jnp.float32),
            pltpu.SemaphoreType.DMA,
        ],
    )
    def gather_kernel(table_hbm, idx_hbm, out_hbm, idx_v, rows_v, sem):
        wid = lax.axis_index("s") * info.num_cores + lax.axis_index("c")
        base = wid * b_per_w
        pltpu.sync_copy(idx_hbm.at[pl.ds(base, b_per_w)], idx_v)
        pltpu.async_copy(table_hbm.at[idx_v], rows_v, sem).wait()
        pltpu.sync_copy(rows_v, out_hbm.at[pl.ds(base, b_per_w)])

    return gather_kernel


def _gather_rows(table, ids):
    return _make_sc_gather()(table, ids)


def _embed_mlp_body(e_ref, pw1_ref, pb1_ref, pw2_ref, pb2_ref, out_ref):
    h1 = jax.nn.gelu(jnp.dot(e_ref[...], pw1_ref[...]) + pb1_ref[...])
    out_ref[...] = jnp.dot(h1, pw2_ref[...]) + pb2_ref[...]


def _embed_mlp(e_all, pw1, pb1r, pw2, pb2r):
    nblk = NIDS_PAD // 256
    return pl.pallas_call(
        _embed_mlp_body,
        grid=(nblk,),
        in_specs=[
            pl.BlockSpec((256, SED), lambda i: (i, 0)),
            pl.BlockSpec((SED, H), lambda i: (0, 0)),
            pl.BlockSpec((1, H), lambda i: (0, 0)),
            pl.BlockSpec((H, H), lambda i: (0, 0)),
            pl.BlockSpec((1, H), lambda i: (0, 0)),
        ],
        out_specs=pl.BlockSpec((256, H), lambda i: (i, 0)),
        out_shape=jax.ShapeDtypeStruct((NIDS_PAD, H), jnp.float32),
    )(e_all, pw1, pb1r, pw2, pb2r)


def _proj_body(qf_ref, bsf_ref, bse_ref, qse_ref, qw_ref, qb_ref, kw_ref,
               kb_ref, vw_ref, vb_ref, gw1a_ref, gw1b_ref, gb1_ref, gw2_ref,
               gb2_ref, v_out, gs_out, gs31_out):
    kv = bsf_ref[0] + bse_ref[0]
    k = jnp.dot(kv, kw_ref[...]) + kb_ref[...]
    v_out[0] = jnp.dot(kv, vw_ref[...]) + vb_ref[...]
    qf = qf_ref[0]
    qc = jnp.mean(qf, axis=0, keepdims=True) + qse_ref[0]
    q = jnp.dot(qf, qw_ref[...]) + qb_ref[...]
    qt = q.T
    g1 = jnp.dot(qc, gw1a_ref[...]) + jnp.dot(kv, gw1b_ref[...]) + gb1_ref[...]
    g = jax.nn.sigmoid(jnp.dot(jax.nn.gelu(g1), gw2_ref[...]) + gb2_ref[...])
    scale = 1.0 / math.sqrt(DH)
    for h in range(HEADS):
        kh = k[:, h * DH:(h + 1) * DH]
        qh_t = qt[h * DH:(h + 1) * DH, :]
        sh_t = jnp.dot(kh, qh_t) * scale * g[:, h:h + 1]
        gs_out[0, :, h * S:(h + 1) * S] = sh_t
        gs31_out[0, :, h:h + 1] = sh_t[:, S - 1:S]


def _proj(qf, bsf, bse, qse, qw, qbr, kw, kbr, vw, vbr, gw1a, gw1b, gb1r,
          gw2, gb2r):
    return pl.pallas_call(
        _proj_body,
        grid=(B,),
        in_specs=[
            pl.BlockSpec((1, S, H), lambda b: (b, 0, 0)),
            pl.BlockSpec((1, NKV, H), lambda b: (b, 0, 0)),
            pl.BlockSpec((1, NKV, H), lambda b: (b, 0, 0)),
            pl.BlockSpec((1, 1, H), lambda b: (b, 0, 0)),
            pl.BlockSpec((H, H), lambda b: (0, 0)),
            pl.BlockSpec((1, H), lambda b: (0, 0)),
            pl.BlockSpec((H, H), lambda b: (0, 0)),
            pl.BlockSpec((1, H), lambda b: (0, 0)),
            pl.BlockSpec((H, H), lambda b: (0, 0)),
            pl.BlockSpec((1, H), lambda b: (0, 0)),
            pl.BlockSpec((H, H), lambda b: (0, 0)),
            pl.BlockSpec((H, H), lambda b: (0, 0)),
            pl.BlockSpec((1, H), lambda b: (0, 0)),
            pl.BlockSpec((H, HEADS), lambda b: (0, 0)),
            pl.BlockSpec((1, HEADS), lambda b: (0, 0)),
        ],
        out_specs=[
            pl.BlockSpec((1, NKV, H), lambda b: (b, 0, 0)),
            pl.BlockSpec((1, NKV, ROWS), lambda b: (b, 0, 0)),
            pl.BlockSpec((1, NKV, HEADS), lambda b: (b, 0, 0)),
        ],
        out_shape=[
            jax.ShapeDtypeStruct((B, NKV, H), jnp.float32),
            jax.ShapeDtypeStruct((B, NKV, ROWS), jnp.float32),
            jax.ShapeDtypeStruct((B, NKV, HEADS), jnp.float32),
        ],
    )(qf, bsf, bse, qse, qw, qbr, kw, kbr, vw, vbr, gw1a, gw1b, gb1r, gw2,
      gb2r)


def _ce_stage(x, d, wantmax_col):
    n = x.shape[0]
    pad = jnp.zeros((d, x.shape[1]), x.dtype)
    up = jnp.concatenate([x[d:], pad], axis=0)
    dn = jnp.concatenate([pad, x[:n - d]], axis=0)
    pos = lax.broadcasted_iota(jnp.int32, (n, 1), 0)
    is_lower = (pos & d) == 0
    p = jnp.where(is_lower, up, dn)
    take_self = (x >= p) == wantmax_col
    return jnp.where(take_self, x, p)


def _bitonic_top128_desc(x):
    n = x.shape[0]
    pos = lax.broadcasted_iota(jnp.int32, (n, 1), 0)
    posb = pos & 127
    even_blk = (pos & 128) == 0
    for size in (2, 4, 8, 16, 32, 64, 128):
        dir_desc = ((posb & size) == 0) == even_blk
        d = size // 2
        while d >= 1:
            wantmax = dir_desc == ((pos & d) == 0)
            x = _ce_stage(x, d, wantmax)
            d //= 2
    y = jnp.concatenate([jnp.maximum(x[0:128], x[128:256]),
                         jnp.maximum(x[256:384], x[384:512])], axis=0)
    posy = lax.broadcasted_iota(jnp.int32, (256, 1), 0)
    even_y = (posy & 128) == 0
    d = 64
    while d >= 1:
        wantmax = even_y == ((posy & d) == 0)
        y = _ce_stage(y, d, wantmax)
        d //= 2
    z = jnp.maximum(y[0:128], y[128:256])
    posz = lax.broadcasted_iota(jnp.int32, (128, 1), 0)
    d = 64
    while d >= 1:
        wantmax = (posz & d) == 0
        z = _ce_stage(z, d, wantmax)
        d //= 2
    return z
def _attend_body(gs_ref, gs31_ref, v_ref, qf_ref, ow_ref, ob_ref, lng_ref,
                 lnb_ref, y_out, attn_out, idx31_out, idx31_ref, ctx_ref):
    sc = gs_ref[0]
    sc31 = gs31_ref[0]
    iota31 = lax.broadcasted_iota(jnp.int32, (NKV, HEADS), 0)

    st = _bitonic_top128_desc(sc)

    def step(j, carry):
        pv31, pi31 = carry
        rem = (sc31 < pv31) | ((sc31 == pv31) & (iota31 > pi31))
        cur = jnp.max(jnp.where(rem, sc31, -jnp.inf), axis=0, keepdims=True)
        hit = rem & (sc31 == cur)
        idxv = jnp.min(jnp.where(hit, iota31, jnp.int32(1 << 30)), axis=0,
                       keepdims=True)
        idx31_ref[pl.ds(j, 1), :] = idxv
        return cur, idxv

    lax.fori_loop(0, TOPK, step,
                  (jnp.full((1, HEADS), jnp.inf, jnp.float32),
                   jnp.full((1, HEADS), -1, jnp.int32)))
    v0 = st[0:1, :]
    e = jnp.exp(st - v0)
    denom = jnp.sum(e, axis=0, keepdims=True)
    attn_out[0] = (e / denom).T
    idx31_out[0] = idx31_ref[...].T

    th = st[TOPK - 1:TOPK, :]
    gt = sc > th
    eqth = (sc == th).astype(jnp.float32)
    k_need = TOPK - jnp.sum(gt.astype(jnp.float32), axis=0, keepdims=True)
    rank = eqth
    for shift in (1, 2, 4, 8, 16, 32, 64, 128, 256):
        shifted = jnp.concatenate(
            [jnp.zeros((shift, ROWS), jnp.float32), rank[:NKV - shift]],
            axis=0)
        rank = rank + shifted
    sel = gt | ((eqth > 0.0) & (rank <= k_need))
    p_t = jnp.where(sel, jnp.exp(sc - v0), 0.0) / denom
    p = p_t.T
    vv = v_ref[0]
    for h in range(HEADS):
        ph = p[h * S:(h + 1) * S, :]
        vh = vv[:, h * DH:(h + 1) * DH]
        ctx_ref[:, h * DH:(h + 1) * DH] = jnp.dot(ph, vh)

    out = jnp.dot(ctx_ref[...], ow_ref[...]) + ob_ref[...]
    x = qf_ref[0] + out
    mu = jnp.mean(x, axis=1, keepdims=True)
    var = jnp.mean((x - mu) ** 2, axis=1, keepdims=True)
    y_out[0] = (x - mu) * lax.rsqrt(var + 1e-5) * lng_ref[...] + lnb_ref[...]


def _attend(gs, gs31, v, qf, ow, obr, lngr, lnbr):
    return pl.pallas_call(
        _attend_body,
        grid=(B,),
        in_specs=[
            pl.BlockSpec((1, NKV, ROWS), lambda b: (b, 0, 0)),
            pl.BlockSpec((1, NKV, HEADS), lambda b: (b, 0, 0)),
            pl.BlockSpec((1, NKV, H), lambda b: (b, 0, 0)),
            pl.BlockSpec((1, S, H), lambda b: (b, 0, 0)),
            pl.BlockSpec((H, H), lambda b: (0, 0)),
            pl.BlockSpec((1, H), lambda b: (0, 0)),
            pl.BlockSpec((1, H), lambda b: (0, 0)),
            pl.BlockSpec((1, H), lambda b: (0, 0)),
        ],
        out_specs=[
            pl.BlockSpec((1, S, H), lambda b: (b, 0, 0)),
            pl.BlockSpec((1, ROWS, TOPK), lambda b: (b, 0, 0)),
            pl.BlockSpec((1, HEADS, TOPK), lambda b: (b, 0, 0)),
        ],
        out_shape=[
            jax.ShapeDtypeStruct((B, S, H), jnp.float32),
            jax.ShapeDtypeStruct((B, ROWS, TOPK), jnp.float32),
            jax.ShapeDtypeStruct((B, HEADS, TOPK), jnp.int32),
        ],
        scratch_shapes=[
            pltpu.VMEM((TOPK, HEADS), jnp.int32),
            pltpu.VMEM((S, H), jnp.float32),
        ],
    )(gs, gs31, v, qf, ow, obr, lngr, lnbr)


def kernel(query_features, query_stock_ids, batch_stock_ids,
           batch_stock_features, stock_table, pw1, pb1, pw2, pb2, qw, qb, kw,
           kb, vw, vb, ow, ob, gw1, gb1, gw2, gb2, ln_g, ln_b):
    ids_all = jnp.concatenate([
        query_stock_ids.astype(jnp.int32),
        batch_stock_ids.reshape(-1).astype(jnp.int32),
        jnp.zeros((NIDS_PAD - NIDS,), jnp.int32),
    ])
    e_all = _gather_rows(stock_table, ids_all)
    emb = _embed_mlp(e_all, pw1, pb1.reshape(1, H), pw2, pb2.reshape(1, H))
    qse = emb[:B].reshape(B, 1, H)
    bse = emb[B:B + B * NKV].reshape(B, NKV, H)

    v_all, gs, gs31 = _proj(
        query_features, batch_stock_features, bse, qse,
        qw, qb.reshape(1, H), kw, kb.reshape(1, H), vw, vb.reshape(1, H),
        gw1[:H], gw1[H:], gb1.reshape(1, H), gw2, gb2.reshape(1, HEADS))

    y, attn, attended = _attend(
        gs, gs31, v_all, query_features, ow, ob.reshape(1, H),
        ln_g.reshape(1, H), ln_b.reshape(1, H))

    attn = attn.reshape(B, HEADS, S, TOPK)
    return y, attn, attended


# speedup vs baseline: 34.8478x; 1.5498x over previous
import functools
import math

import jax
import jax.numpy as jnp
from jax import lax
from jax.experimental import pallas as pl
from jax.experimental.pallas import tpu as pltpu
from jax.experimental.pallas import tpu_sc as plsc

B, S, H = 8, 32, 1024
HEADS, DH = 16, 64
NSTOCK, SED = 6000, 256
NKV, TOPK = 512, 128

NIDS = B + B * NKV
NIDS_PAD = 4352
ROWS = HEADS * S


@functools.lru_cache(maxsize=1)
def _make_sc_gather():
    info = plsc.get_sparse_core_info()
    nw = info.num_cores * info.num_subcores
    b_per_w = NIDS_PAD // nw
    mesh = plsc.VectorSubcoreMesh(core_axis_name="c", subcore_axis_name="s")

    @functools.partial(
        pl.kernel,
        mesh=mesh,
        out_type=jax.ShapeDtypeStruct((NIDS_PAD, SED), jnp.float32),
        scratch_types=[
            pltpu.VMEM((b_per_w,), jnp.int32),
            pltpu.VMEM((b_per_w, SED), jnp.float32),
            pltpu.SemaphoreType.DMA,
        ],
    )
    def gather_kernel(table_hbm, idx_hbm, out_hbm, idx_v, rows_v, sem):
        wid = lax.axis_index("s") * info.num_cores + lax.axis_index("c")
        base = wid * b_per_w
        pltpu.sync_copy(idx_hbm.at[pl.ds(base, b_per_w)], idx_v)
        pltpu.async_copy(table_hbm.at[idx_v], rows_v, sem).wait()
        pltpu.sync_copy(rows_v, out_hbm.at[pl.ds(base, b_per_w)])

    return gather_kernel


def _gather_rows(table, ids):
    return _make_sc_gather()(table, ids)


def _embed_mlp_body(e_ref, pw1_ref, pb1_ref, pw2_ref, pb2_ref, out_ref):
    h1 = jax.nn.gelu(jnp.dot(e_ref[...], pw1_ref[...]) + pb1_ref[...])
    out_ref[...] = jnp.dot(h1, pw2_ref[...]) + pb2_ref[...]


def _embed_mlp(e_all, pw1, pb1r, pw2, pb2r):
    nblk = NIDS_PAD // 256
    return pl.pallas_call(
        _embed_mlp_body,
        grid=(nblk,),
        in_specs=[
            pl.BlockSpec((256, SED), lambda i: (i, 0)),
            pl.BlockSpec((SED, H), lambda i: (0, 0)),
            pl.BlockSpec((1, H), lambda i: (0, 0)),
            pl.BlockSpec((H, H), lambda i: (0, 0)),
            pl.BlockSpec((1, H), lambda i: (0, 0)),
        ],
        out_specs=pl.BlockSpec((256, H), lambda i: (i, 0)),
        out_shape=jax.ShapeDtypeStruct((NIDS_PAD, H), jnp.float32),
    )(e_all, pw1, pb1r, pw2, pb2r)


def _proj_body(qf_ref, bsf_ref, bse_ref, qse_ref, qw_ref, qb_ref, kw_ref,
               kb_ref, vw_ref, vb_ref, gw1a_ref, gw1b_ref, gb1_ref, gw2_ref,
               gb2_ref, v_out, gs_out, gs31_out):
    kv = bsf_ref[0] + bse_ref[0]
    k = jnp.dot(kv, kw_ref[...]) + kb_ref[...]
    v_out[0] = jnp.dot(kv, vw_ref[...]) + vb_ref[...]
    qf = qf_ref[0]
    qc = jnp.mean(qf, axis=0, keepdims=True) + qse_ref[0]
    q = jnp.dot(qf, qw_ref[...]) + qb_ref[...]
    qt = q.T
    g1 = jnp.dot(qc, gw1a_ref[...]) + jnp.dot(kv, gw1b_ref[...]) + gb1_ref[...]
    g = jax.nn.sigmoid(jnp.dot(jax.nn.gelu(g1), gw2_ref[...]) + gb2_ref[...])
    scale = 1.0 / math.sqrt(DH)
    for h in range(HEADS):
        kh = k[:, h * DH:(h + 1) * DH]
        qh_t = qt[h * DH:(h + 1) * DH, :]
        sh_t = jnp.dot(kh, qh_t) * scale * g[:, h:h + 1]
        gs_out[0, :, h * S:(h + 1) * S] = sh_t
        gs31_out[0, :, h:h + 1] = sh_t[:, S - 1:S]


def _proj(qf, bsf, bse, qse, qw, qbr, kw, kbr, vw, vbr, gw1a, gw1b, gb1r,
          gw2, gb2r):
    return pl.pallas_call(
        _proj_body,
        grid=(B,),
        in_specs=[
            pl.BlockSpec((1, S, H), lambda b: (b, 0, 0)),
            pl.BlockSpec((1, NKV, H), lambda b: (b, 0, 0)),
            pl.BlockSpec((1, NKV, H), lambda b: (b, 0, 0)),
            pl.BlockSpec((1, 1, H), lambda b: (b, 0, 0)),
            pl.BlockSpec((H, H), lambda b: (0, 0)),
            pl.BlockSpec((1, H), lambda b: (0, 0)),
            pl.BlockSpec((H, H), lambda b: (0, 0)),
            pl.BlockSpec((1, H), lambda b: (0, 0)),
            pl.BlockSpec((H, H), lambda b: (0, 0)),
            pl.BlockSpec((1, H), lambda b: (0, 0)),
            pl.BlockSpec((H, H), lambda b: (0, 0)),
            pl.BlockSpec((H, H), lambda b: (0, 0)),
            pl.BlockSpec((1, H), lambda b: (0, 0)),
            pl.BlockSpec((H, HEADS), lambda b: (0, 0)),
            pl.BlockSpec((1, HEADS), lambda b: (0, 0)),
        ],
        out_specs=[
            pl.BlockSpec((1, NKV, H), lambda b: (b, 0, 0)),
            pl.BlockSpec((1, NKV, ROWS), lambda b: (b, 0, 0)),
            pl.BlockSpec((1, NKV, HEADS), lambda b: (b, 0, 0)),
        ],
        out_shape=[
            jax.ShapeDtypeStruct((B, NKV, H), jnp.float32),
            jax.ShapeDtypeStruct((B, NKV, ROWS), jnp.float32),
            jax.ShapeDtypeStruct((B, NKV, HEADS), jnp.float32),
        ],
    )(qf, bsf, bse, qse, qw, qbr, kw, kbr, vw, vbr, gw1a, gw1b, gb1r, gw2,
      gb2r)


def _ce_stage(x, d, wantmax_col):
    n = x.shape[0]
    pad = jnp.zeros((d, x.shape[1]), x.dtype)
    up = jnp.concatenate([x[d:], pad], axis=0)
    dn = jnp.concatenate([pad, x[:n - d]], axis=0)
    pos = lax.broadcasted_iota(jnp.int32, (n, 1), 0)
    is_lower = (pos & d) == 0
    p = jnp.where(is_lower, up, dn)
    take_self = (x >= p) == wantmax_col
    return jnp.where(take_self, x, p)


def _bitonic_top128_desc(x):
    n = x.shape[0]
    pos = lax.broadcasted_iota(jnp.int32, (n, 1), 0)
    posb = pos & 127
    even_blk = (pos & 128) == 0
    for size in (2, 4, 8, 16, 32, 64, 128):
        dir_desc = ((posb & size) == 0) == even_blk
        d = size // 2
        while d >= 1:
            wantmax = dir_desc == ((pos & d) == 0)
            x = _ce_stage(x, d, wantmax)
            d //= 2
    y = jnp.concatenate([jnp.maximum(x[0:128], x[128:256]),
                         jnp.maximum(x[256:384], x[384:512])], axis=0)
    posy = lax.broadcasted_iota(jnp.int32, (256, 1), 0)
    even_y = (posy & 128) == 0
    d = 64
    while d >= 1:
        wantmax = even_y == ((posy & d) == 0)
        y = _ce_stage(y, d, wantmax)
        d //= 2
    z = jnp.maximum(y[0:128], y[128:256])
    posz = lax.broadcasted_iota(jnp.int32, (128, 1), 0)
    d = 64
    while d >= 1:
        wantmax = (posz & d) == 0
        z = _ce_stage(z, d, wantmax)
        d //= 2
    return z


def _attend_body(gs_ref, v_ref, qf_ref, ow_ref, ob_ref, lng_ref,
                 lnb_ref, y_out, attn_out, ctx_ref):
    sc = gs_ref[0]

    st = _bitonic_top128_desc(sc)

    v0 = st[0:1, :]
    e = jnp.exp(st - v0)
    denom = jnp.sum(e, axis=0, keepdims=True)
    attn_out[0] = (e / denom).T

    th = st[TOPK - 1:TOPK, :]
    gt = sc > th
    eqth = (sc == th).astype(jnp.float32)
    k_need = TOPK - jnp.sum(gt.astype(jnp.float32), axis=0, keepdims=True)
    rank = eqth
    for shift in (1, 2, 4, 8, 16, 32, 64, 128, 256):
        shifted = jnp.concatenate(
            [jnp.zeros((shift, ROWS), jnp.float32), rank[:NKV - shift]],
            axis=0)
        rank = rank + shifted
    sel = gt | ((eqth > 0.0) & (rank <= k_need))
    p_t = jnp.where(sel, jnp.exp(sc - v0), 0.0) / denom
    p = p_t.T
    vv = v_ref[0]
    for h in range(HEADS):
        ph = p[h * S:(h + 1) * S, :]
        vh = vv[:, h * DH:(h + 1) * DH]
        ctx_ref[:, h * DH:(h + 1) * DH] = jnp.dot(ph, vh)

    out = jnp.dot(ctx_ref[...], ow_ref[...]) + ob_ref[...]
    x = qf_ref[0] + out
    mu = jnp.mean(x, axis=1, keepdims=True)
    var = jnp.mean((x - mu) ** 2, axis=1, keepdims=True)
    y_out[0] = (x - mu) * lax.rsqrt(var + 1e-5) * lng_ref[...] + lnb_ref[...]


def _attend(gs, v, qf, ow, obr, lngr, lnbr):
    return pl.pallas_call(
        _attend_body,
        grid=(B,),
        in_specs=[
            pl.BlockSpec((1, NKV, ROWS), lambda b: (b, 0, 0)),
            pl.BlockSpec((1, NKV, H), lambda b: (b, 0, 0)),
            pl.BlockSpec((1, S, H), lambda b: (b, 0, 0)),
            pl.BlockSpec((H, H), lambda b: (0, 0)),
            pl.BlockSpec((1, H), lambda b: (0, 0)),
            pl.BlockSpec((1, H), lambda b: (0, 0)),
            pl.BlockSpec((1, H), lambda b: (0, 0)),
        ],
        out_specs=[
            pl.BlockSpec((1, S, H), lambda b: (b, 0, 0)),
            pl.BlockSpec((1, ROWS, TOPK), lambda b: (b, 0, 0)),
        ],
        out_shape=[
            jax.ShapeDtypeStruct((B, S, H), jnp.float32),
            jax.ShapeDtypeStruct((B, ROWS, TOPK), jnp.float32),
        ],
        scratch_shapes=[
            pltpu.VMEM((S, H), jnp.float32),
        ],
    )(gs, v, qf, ow, obr, lngr, lnbr)


def _attended_body(gs31_ref, idx_out, sc_ref, idxs_ref):
    cols = B * HEADS
    for b in range(B):
        sc_ref[:, b * HEADS:(b + 1) * HEADS] = gs31_ref[b]
    sc31 = sc_ref[...]
    iota31 = lax.broadcasted_iota(jnp.int32, (NKV, cols), 0)

    def step(j, carry):
        pv, pi = carry
        rem = (sc31 < pv) | ((sc31 == pv) & (iota31 > pi))
        cur = jnp.max(jnp.where(rem, sc31, -jnp.inf), axis=0, keepdims=True)
        hit = rem & (sc31 == cur)
        idxv = jnp.min(jnp.where(hit, iota31, jnp.int32(1 << 30)), axis=0,
                       keepdims=True)
        idxs_ref[pl.ds(j, 1), :] = idxv
        return cur, idxv

    lax.fori_loop(0, TOPK, step,
                  (jnp.full((1, cols), jnp.inf, jnp.float32),
                   jnp.full((1, cols), -1, jnp.int32)))
    idx_out[...] = idxs_ref[...].T


def _attended(gs31):
    return pl.pallas_call(
        _attended_body,
        out_shape=jax.ShapeDtypeStruct((B * HEADS, TOPK), jnp.int32),
        scratch_shapes=[pltpu.VMEM((NKV, B * HEADS), jnp.float32),
                        pltpu.VMEM((TOPK, B * HEADS), jnp.int32)],
    )(gs31)


def kernel(query_features, query_stock_ids, batch_stock_ids,
           batch_stock_features, stock_table, pw1, pb1, pw2, pb2, qw, qb, kw,
           kb, vw, vb, ow, ob, gw1, gb1, gw2, gb2, ln_g, ln_b):
    ids_all = jnp.concatenate([
        query_stock_ids.astype(jnp.int32),
        batch_stock_ids.reshape(-1).astype(jnp.int32),
        jnp.zeros((NIDS_PAD - NIDS,), jnp.int32),
    ])
    e_all = _gather_rows(stock_table, ids_all)
    emb = _embed_mlp(e_all, pw1, pb1.reshape(1, H), pw2, pb2.reshape(1, H))
    qse = emb[:B].reshape(B, 1, H)
    bse = emb[B:B + B * NKV].reshape(B, NKV, H)

    v_all, gs, gs31 = _proj(
        query_features, batch_stock_features, bse, qse,
        qw, qb.reshape(1, H), kw, kb.reshape(1, H), vw, vb.reshape(1, H),
        gw1[:H], gw1[H:], gb1.reshape(1, H), gw2, gb2.reshape(1, HEADS))

    y, attn = _attend(
        gs, v_all, query_features, ow, ob.reshape(1, H),
        ln_g.reshape(1, H), ln_b.reshape(1, H))
    attended = _attended(gs31).reshape(B, HEADS, TOPK)

    attn = attn.reshape(B, HEADS, S, TOPK)
    return y, attn, attended
